# baseline (speedup 1.0000x reference)
import functools

import jax
import jax.numpy as jnp
from jax import lax
from jax.experimental import pallas as pl
from jax.experimental.pallas import tpu as pltpu
from jax.experimental.pallas import tpu_sc as plsc

VOCAB = 100000
D = 128
B = 4096
L = 50
EPS = 1e-5

NC = 2
NS = 16
NW = NC * NS
ROWS_PER_W = B // NW
CHUNKS = ROWS_PER_W
NBUF = 8
NV = D // 16


def _pool_sc(x, emb_table):
    @functools.partial(
        pl.kernel,
        out_type=jax.ShapeDtypeStruct((B, D), jnp.float32),
        mesh=plsc.VectorSubcoreMesh(core_axis_name="c", subcore_axis_name="s"),
        scratch_types=(
            [pltpu.VMEM((ROWS_PER_W, L), jnp.int32)]
            + [pltpu.VMEM((L, D), jnp.float32) for _ in range(NBUF)]
            + [pltpu.VMEM((ROWS_PER_W, D), jnp.float32)]
            + [pltpu.SemaphoreType.DMA for _ in range(NBUF)]
        ),
    )
    def k(x_hbm, table_hbm, out_hbm, idx_v, *rest):
        bufs = rest[:NBUF]
        pooled_v = rest[NBUF]
        sems = rest[NBUF + 1:]
        wid = lax.axis_index("s") * NC + lax.axis_index("c")
        pltpu.sync_copy(x_hbm.at[pl.ds(wid * ROWS_PER_W, ROWS_PER_W)], idx_v)
        for p in range(NBUF):
            pltpu.make_async_copy(
                table_hbm.at[idx_v.at[p]], bufs[p], sems[p]).start()

        UNROLL = 25

        def reduce_chunk(c, rows_v):
            def blk(t, acc):
                base = t * UNROLL
                for j in range(UNROLL):
                    acc = tuple(
                        acc[k] + rows_v[base + j, pl.ds(k * 16, 16)]
                        for k in range(NV)
                    )
                return acc
            acc0 = tuple(jnp.zeros((16,), jnp.float32) for _ in range(NV))
            accs = lax.fori_loop(0, L // UNROLL, blk, acc0)
            for k in range(NV):
                pooled_v[c, pl.ds(k * 16, 16)] = accs[k]

        def body(i, carry):
            c = NBUF * i
            for p in range(NBUF):
                cp = c + p
                pltpu.make_async_copy(
                    table_hbm.at[idx_v.at[cp]], bufs[p], sems[p]).wait()
                reduce_chunk(cp, bufs[p])
                nxt = cp + NBUF

                @pl.when(nxt < CHUNKS)
                def _(p=p, nxt=nxt):
                    pltpu.make_async_copy(
                        table_hbm.at[idx_v.at[nxt]], bufs[p], sems[p]).start()
            return carry

        lax.fori_loop(0, CHUNKS // NBUF, body, 0)
        pltpu.sync_copy(pooled_v, out_hbm.at[pl.ds(wid * ROWS_PER_W, ROWS_PER_W)])

    return k(x, emb_table)


def _dense_kernel(pooled_ref, w_ref, b_ref, bng_ref, bnb_ref, lng_ref, lnb_ref,
                  out_ref):
    p = pooled_ref[...] * (1.0 / L)
    h = lax.dot_general(
        p, w_ref[...], (((1,), (1,)), ((), ())),
        preferred_element_type=jnp.float32,
        precision=lax.Precision.HIGHEST,
    ) + b_ref[...]
    mu = jnp.mean(h, axis=0, keepdims=True)
    var = jnp.mean((h - mu) * (h - mu), axis=0, keepdims=True)
    h = (h - mu) * lax.rsqrt(var + EPS) * bng_ref[...] + bnb_ref[...]
    m = jnp.mean(h, axis=1, keepdims=True)
    v = jnp.mean((h - m) * (h - m), axis=1, keepdims=True)
    out_ref[...] = (h - m) * lax.rsqrt(v + EPS) * lng_ref[...] + lnb_ref[...]


def _dense_tc(pooled, W, b, bn_gamma, bn_beta, ln_gamma, ln_beta):
    return pl.pallas_call(
        _dense_kernel,
        out_shape=jax.ShapeDtypeStruct((B, D), jnp.float32),
    )(pooled, W, b.reshape(1, D), bn_gamma.reshape(1, D),
      bn_beta.reshape(1, D), ln_gamma.reshape(1, D), ln_beta.reshape(1, D))


@jax.jit
def kernel(x, emb_table, W, b, bn_gamma, bn_beta, ln_gamma, ln_beta):
    pooled = _pool_sc(x.astype(jnp.int32), emb_table)
    return _dense_tc(pooled, W, b, bn_gamma, bn_beta, ln_gamma, ln_beta)


# speedup vs baseline: 1.4954x; 1.4954x over previous
import functools

import jax
import jax.numpy as jnp
from jax import lax
from jax.experimental import pallas as pl
from jax.experimental.pallas import tpu as pltpu
from jax.experimental.pallas import tpu_sc as plsc

VOCAB = 100000
D = 128
B = 4096
L = 50
EPS = 1e-5

NC = 2
NS = 16
NW = NC * NS
ROWS_PER_W = B // NW
CHUNKS = ROWS_PER_W
NBUF = 8
NV = D // 16


def _pool_sc(x, emb_table):
    @functools.partial(
        pl.kernel,
        out_type=jax.ShapeDtypeStruct((B, D), jnp.float32),
        mesh=plsc.VectorSubcoreMesh(core_axis_name="c", subcore_axis_name="s"),
        scratch_types=(
            [pltpu.VMEM((ROWS_PER_W, L), jnp.int32)]
            + [pltpu.VMEM((L, D), jnp.float32) for _ in range(NBUF)]
            + [pltpu.VMEM((ROWS_PER_W, D), jnp.float32)]
            + [pltpu.SemaphoreType.DMA for _ in range(NBUF)]
        ),
    )
    def k(x_hbm, table_hbm, out_hbm, idx_v, *rest):
        bufs = rest[:NBUF]
        pooled_v = rest[NBUF]
        sems = rest[NBUF + 1:]
        wid = lax.axis_index("s") * NC + lax.axis_index("c")
        pltpu.sync_copy(x_hbm.at[pl.ds(wid * ROWS_PER_W, ROWS_PER_W)], idx_v)
        for p in range(NBUF):
            pltpu.make_async_copy(
                table_hbm.at[idx_v.at[p]], bufs[p], sems[p]).start()

        UNROLL = 5

        def reduce_chunk(c, rows_v):
            def blk(t, acc):
                base = t * UNROLL
                for j in range(UNROLL):
                    acc = tuple(
                        acc[k] + rows_v[base + j, pl.ds(k * 16, 16)]
                        for k in range(NV)
                    )
                return acc
            acc0 = tuple(jnp.zeros((16,), jnp.float32) for _ in range(NV))
            accs = lax.fori_loop(0, L // UNROLL, blk, acc0)
            for k in range(NV):
                pooled_v[c, pl.ds(k * 16, 16)] = accs[k]

        def body(i, carry):
            c = NBUF * i
            for p in range(NBUF):
                cp = c + p
                pltpu.make_async_copy(
                    table_hbm.at[idx_v.at[cp]], bufs[p], sems[p]).wait()
                reduce_chunk(cp, bufs[p])
                nxt = cp + NBUF

                @pl.when(nxt < CHUNKS)
                def _(p=p, nxt=nxt):
                    pltpu.make_async_copy(
                        table_hbm.at[idx_v.at[nxt]], bufs[p], sems[p]).start()
            return carry

        lax.fori_loop(0, CHUNKS // NBUF, body, 0)
        pltpu.sync_copy(pooled_v, out_hbm.at[pl.ds(wid * ROWS_PER_W, ROWS_PER_W)])

    return k(x, emb_table)


def _dense_kernel(pooled_ref, w_ref, b_ref, bng_ref, bnb_ref, lng_ref, lnb_ref,
                  out_ref):
    p = pooled_ref[...] * (1.0 / L)
    h = lax.dot_general(
        p, w_ref[...], (((1,), (1,)), ((), ())),
        preferred_element_type=jnp.float32,
        precision=lax.Precision.HIGHEST,
    ) + b_ref[...]
    mu = jnp.mean(h, axis=0, keepdims=True)
    var = jnp.mean((h - mu) * (h - mu), axis=0, keepdims=True)
    h = (h - mu) * lax.rsqrt(var + EPS) * bng_ref[...] + bnb_ref[...]
    m = jnp.mean(h, axis=1, keepdims=True)
    v = jnp.mean((h - m) * (h - m), axis=1, keepdims=True)
    out_ref[...] = (h - m) * lax.rsqrt(v + EPS) * lng_ref[...] + lnb_ref[...]


def _dense_tc(pooled, W, b, bn_gamma, bn_beta, ln_gamma, ln_beta):
    return pl.pallas_call(
        _dense_kernel,
        out_shape=jax.ShapeDtypeStruct((B, D), jnp.float32),
    )(pooled, W, b.reshape(1, D), bn_gamma.reshape(1, D),
      bn_beta.reshape(1, D), ln_gamma.reshape(1, D), ln_beta.reshape(1, D))


@jax.jit
def kernel(x, emb_table, W, b, bn_gamma, bn_beta, ln_gamma, ln_beta):
    pooled = _pool_sc(x.astype(jnp.int32), emb_table)
    return _dense_tc(pooled, W, b, bn_gamma, bn_beta, ln_gamma, ln_beta)


# speedup vs baseline: 1.5319x; 1.0245x over previous
import functools

import jax
import jax.numpy as jnp
from jax import lax
from jax.experimental import pallas as pl
from jax.experimental.pallas import tpu as pltpu
from jax.experimental.pallas import tpu_sc as plsc

VOCAB = 100000
D = 128
B = 4096
L = 50
EPS = 1e-5

NC = 2
NS = 16
NW = NC * NS
ROWS_PER_W = B // NW
CHUNKS = ROWS_PER_W
NBUF = 8
NV = D // 16


def _pool_sc(x, emb_table):
    @functools.partial(
        pl.kernel,
        out_type=jax.ShapeDtypeStruct((B, D), jnp.float32),
        mesh=plsc.VectorSubcoreMesh(core_axis_name="c", subcore_axis_name="s"),
        scratch_types=(
            [pltpu.VMEM((ROWS_PER_W, L), jnp.int32)]
            + [pltpu.VMEM((L, D), jnp.float32) for _ in range(NBUF)]
            + [pltpu.VMEM((ROWS_PER_W, D), jnp.float32)]
            + [pltpu.SemaphoreType.DMA for _ in range(NBUF)]
        ),
    )
    def k(x_hbm, table_hbm, out_hbm, idx_v, *rest):
        bufs = rest[:NBUF]
        pooled_v = rest[NBUF]
        sems = rest[NBUF + 1:]
        wid = lax.axis_index("s") * NC + lax.axis_index("c")
        pltpu.sync_copy(x_hbm.at[pl.ds(wid * ROWS_PER_W, ROWS_PER_W)], idx_v)
        for p in range(NBUF):
            pltpu.make_async_copy(
                table_hbm.at[idx_v.at[p]], bufs[p], sems[p]).start()

        UNROLL = 2

        def reduce_chunk(c, rows_v):
            def blk(t, acc):
                base = t * UNROLL
                for j in range(UNROLL):
                    acc = tuple(
                        acc[k] + rows_v[base + j, pl.ds(k * 16, 16)]
                        for k in range(NV)
                    )
                return acc
            acc0 = tuple(jnp.zeros((16,), jnp.float32) for _ in range(NV))
            accs = lax.fori_loop(0, L // UNROLL, blk, acc0)
            for k in range(NV):
                pooled_v[c, pl.ds(k * 16, 16)] = accs[k]

        def body(i, carry):
            c = NBUF * i
            for p in range(NBUF):
                cp = c + p
                pltpu.make_async_copy(
                    table_hbm.at[idx_v.at[cp]], bufs[p], sems[p]).wait()
                reduce_chunk(cp, bufs[p])
                nxt = cp + NBUF

                @pl.when(nxt < CHUNKS)
                def _(p=p, nxt=nxt):
                    pltpu.make_async_copy(
                        table_hbm.at[idx_v.at[nxt]], bufs[p], sems[p]).start()
            return carry

        lax.fori_loop(0, CHUNKS // NBUF, body, 0)
        pltpu.sync_copy(pooled_v, out_hbm.at[pl.ds(wid * ROWS_PER_W, ROWS_PER_W)])

    return k(x, emb_table)


def _dense_kernel(pooled_ref, w_ref, b_ref, bng_ref, bnb_ref, lng_ref, lnb_ref,
                  out_ref):
    p = pooled_ref[...] * (1.0 / L)
    h = lax.dot_general(
        p, w_ref[...], (((1,), (1,)), ((), ())),
        preferred_element_type=jnp.float32,
        precision=lax.Precision.HIGHEST,
    ) + b_ref[...]
    mu = jnp.mean(h, axis=0, keepdims=True)
    var = jnp.mean((h - mu) * (h - mu), axis=0, keepdims=True)
    h = (h - mu) * lax.rsqrt(var + EPS) * bng_ref[...] + bnb_ref[...]
    m = jnp.mean(h, axis=1, keepdims=True)
    v = jnp.mean((h - m) * (h - m), axis=1, keepdims=True)
    out_ref[...] = (h - m) * lax.rsqrt(v + EPS) * lng_ref[...] + lnb_ref[...]


def _dense_tc(pooled, W, b, bn_gamma, bn_beta, ln_gamma, ln_beta):
    return pl.pallas_call(
        _dense_kernel,
        out_shape=jax.ShapeDtypeStruct((B, D), jnp.float32),
    )(pooled, W, b.reshape(1, D), bn_gamma.reshape(1, D),
      bn_beta.reshape(1, D), ln_gamma.reshape(1, D), ln_beta.reshape(1, D))


@jax.jit
def kernel(x, emb_table, W, b, bn_gamma, bn_beta, ln_gamma, ln_beta):
    pooled = _pool_sc(x.astype(jnp.int32), emb_table)
    return _dense_tc(pooled, W, b, bn_gamma, bn_beta, ln_gamma, ln_beta)


# speedup vs baseline: 1.5343x; 1.0015x over previous
import functools

import jax
import jax.numpy as jnp
from jax import lax
from jax.experimental import pallas as pl
from jax.experimental.pallas import tpu as pltpu
from jax.experimental.pallas import tpu_sc as plsc

VOCAB = 100000
D = 128
B = 4096
L = 50
EPS = 1e-5

NC = 2
NS = 16
NW = NC * NS
ROWS_PER_W = B // NW
CHUNKS = ROWS_PER_W
NBUF = 8
NV = D // 16


def _pool_sc(x, emb_table):
    @functools.partial(
        pl.kernel,
        out_type=jax.ShapeDtypeStruct((B, D), jnp.float32),
        mesh=plsc.VectorSubcoreMesh(core_axis_name="c", subcore_axis_name="s"),
        scratch_types=(
            [pltpu.VMEM((ROWS_PER_W, L), jnp.int32)]
            + [pltpu.VMEM((L, D), jnp.float32) for _ in range(NBUF)]
            + [pltpu.VMEM((ROWS_PER_W, D), jnp.float32)]
            + [pltpu.SemaphoreType.DMA for _ in range(NBUF)]
        ),
    )
    def k(x_hbm, table_hbm, out_hbm, idx_v, *rest):
        bufs = rest[:NBUF]
        pooled_v = rest[NBUF]
        sems = rest[NBUF + 1:]
        wid = lax.axis_index("s") * NC + lax.axis_index("c")
        pltpu.sync_copy(x_hbm.at[pl.ds(wid * ROWS_PER_W, ROWS_PER_W)], idx_v)
        for p in range(NBUF):
            pltpu.make_async_copy(
                table_hbm.at[idx_v.at[p]], bufs[p], sems[p]).start()

        UNROLL = 1

        def reduce_chunk(c, rows_v):
            def blk(t, acc):
                base = t * UNROLL
                for j in range(UNROLL):
                    acc = tuple(
                        acc[k] + rows_v[base + j, pl.ds(k * 16, 16)]
                        for k in range(NV)
                    )
                return acc
            acc0 = tuple(jnp.zeros((16,), jnp.float32) for _ in range(NV))
            accs = lax.fori_loop(0, L // UNROLL, blk, acc0)
            for k in range(NV):
                pooled_v[c, pl.ds(k * 16, 16)] = accs[k]

        def body(i, carry):
            c = NBUF * i
            for p in range(NBUF):
                cp = c + p
                pltpu.make_async_copy(
                    table_hbm.at[idx_v.at[cp]], bufs[p], sems[p]).wait()
                reduce_chunk(cp, bufs[p])
                nxt = cp + NBUF

                @pl.when(nxt < CHUNKS)
                def _(p=p, nxt=nxt):
                    pltpu.make_async_copy(
                        table_hbm.at[idx_v.at[nxt]], bufs[p], sems[p]).start()
            return carry

        lax.fori_loop(0, CHUNKS // NBUF, body, 0)
        pltpu.sync_copy(pooled_v, out_hbm.at[pl.ds(wid * ROWS_PER_W, ROWS_PER_W)])

    return k(x, emb_table)


def _dense_kernel(pooled_ref, w_ref, b_ref, bng_ref, bnb_ref, lng_ref, lnb_ref,
                  out_ref):
    p = pooled_ref[...] * (1.0 / L)
    h = lax.dot_general(
        p, w_ref[...], (((1,), (1,)), ((), ())),
        preferred_element_type=jnp.float32,
        precision=lax.Precision.HIGHEST,
    ) + b_ref[...]
    mu = jnp.mean(h, axis=0, keepdims=True)
    var = jnp.mean((h - mu) * (h - mu), axis=0, keepdims=True)
    h = (h - mu) * lax.rsqrt(var + EPS) * bng_ref[...] + bnb_ref[...]
    m = jnp.mean(h, axis=1, keepdims=True)
    v = jnp.mean((h - m) * (h - m), axis=1, keepdims=True)
    out_ref[...] = (h - m) * lax.rsqrt(v + EPS) * lng_ref[...] + lnb_ref[...]


def _dense_tc(pooled, W, b, bn_gamma, bn_beta, ln_gamma, ln_beta):
    return pl.pallas_call(
        _dense_kernel,
        out_shape=jax.ShapeDtypeStruct((B, D), jnp.float32),
    )(pooled, W, b.reshape(1, D), bn_gamma.reshape(1, D),
      bn_beta.reshape(1, D), ln_gamma.reshape(1, D), ln_beta.reshape(1, D))


@jax.jit
def kernel(x, emb_table, W, b, bn_gamma, bn_beta, ln_gamma, ln_beta):
    pooled = _pool_sc(x.astype(jnp.int32), emb_table)
    return _dense_tc(pooled, W, b, bn_gamma, bn_beta, ln_gamma, ln_beta)
